# baseline (speedup 1.0000x reference)
import functools

import numpy as np
import jax
import jax.numpy as jnp
from jax import lax
from jax.experimental import pallas as pl
from jax.experimental.pallas import tpu as pltpu
from jax.experimental.pallas import tpu_sc as plsc

VOCAB = 100000
D_MODEL = 1024
MAX_LEN = 2048
BATCH = 4
SEQ = 2048

NC, NS = 2, 16
NW = NC * NS
TOTAL = BATCH * SEQ
PER_W = TOTAL // NW
CHUNK = 16
N_CHUNKS = PER_W // CHUNK
NR = 3
NO = 3
SCALE = float(D_MODEL) ** 0.5


def _make_pe(max_len, d_model):
    pe = np.zeros((max_len, d_model), dtype=np.float32)
    position = np.arange(0, max_len, dtype=np.float32)[:, None]
    div_term = np.exp(
        np.arange(0, d_model, 2, dtype=np.float32) * -(np.log(10000.0) / d_model))
    pe[:, 0::2] = np.sin(position * div_term)
    pe[:, 1::2] = np.cos(position * div_term)
    return pe


_PE = _make_pe(MAX_LEN, D_MODEL)


def _sc_embed(x_flat, table, pe):
    mesh = plsc.VectorSubcoreMesh(core_axis_name="c", subcore_axis_name="s")

    @functools.partial(
        pl.kernel,
        out_type=jax.ShapeDtypeStruct((TOTAL, D_MODEL), jnp.float32),
        mesh=mesh,
        scratch_types=[
            pltpu.VMEM((PER_W,), jnp.int32),
            [pltpu.VMEM((CHUNK, D_MODEL), jnp.float32) for _ in range(NR)],
            [pltpu.VMEM((CHUNK, D_MODEL), jnp.float32) for _ in range(NO)],
            [pltpu.SemaphoreType.DMA for _ in range(NR)],
            [pltpu.SemaphoreType.DMA for _ in range(NO)],
            [pltpu.SemaphoreType.DMA for _ in range(NO)],
        ],
    )
    def k(x_hbm, table_hbm, pe_hbm, out_hbm,
          idx_v, rows, obufs, gsems, psems, osems):
        wid = lax.axis_index("s") * NC + lax.axis_index("c")
        base = wid * PER_W
        s0 = base % SEQ

        pltpu.sync_copy(x_hbm.at[pl.ds(base, PER_W)], idx_v)

        def fire_gather(c):
            pltpu.async_copy(
                table_hbm.at[idx_v.at[pl.ds(c * CHUNK, CHUNK)]],
                rows[c % NR], gsems[c % NR])

        def wait_gather(c):
            pltpu.make_async_copy(
                table_hbm.at[idx_v.at[pl.ds(c * CHUNK, CHUNK)]],
                rows[c % NR], gsems[c % NR]).wait()

        def fire_pe(c):
            pltpu.async_copy(
                pe_hbm.at[pl.ds(s0 + c * CHUNK, CHUNK)],
                obufs[c % NO], psems[c % NO])

        def wait_pe(c):
            pltpu.make_async_copy(
                pe_hbm.at[pl.ds(s0 + c * CHUNK, CHUNK)],
                obufs[c % NO], psems[c % NO]).wait()

        def fire_out(c):
            pltpu.async_copy(
                obufs[c % NO], out_hbm.at[pl.ds(base + c * CHUNK, CHUNK)],
                osems[c % NO])

        def wait_out(c):
            pltpu.make_async_copy(
                obufs[c % NO], out_hbm.at[pl.ds(base + c * CHUNK, CHUNK)],
                osems[c % NO]).wait()

        fire_gather(0)
        fire_gather(1)
        fire_pe(0)

        for c in range(N_CHUNKS):
            if c + 1 < N_CHUNKS:
                if c >= 2:
                    wait_out(c - 2)
                fire_pe(c + 1)
            wait_gather(c)
            wait_pe(c)

            b, o = c % NR, c % NO

            @plsc.parallel_loop(0, CHUNK)
            def row_body(r):
                for q in range(D_MODEL // 16):
                    sl = pl.ds(q * 16, 16)
                    plsc.addupdate(obufs[o].at[r, sl], rows[b][r, sl] * SCALE)

            fire_out(c)
            if c + 2 < N_CHUNKS:
                fire_gather(c + 2)

        for c in range(N_CHUNKS - 3, N_CHUNKS):
            wait_out(c)

    return k(x_flat, table, pe)


def kernel(x, table):
    x_flat = jnp.reshape(x, (TOTAL,)).astype(jnp.int32)
    out = _sc_embed(x_flat, table, _PE)
    return jnp.reshape(out, (BATCH, SEQ, D_MODEL))


# speedup vs baseline: 1.0910x; 1.0910x over previous
import functools

import numpy as np
import jax
import jax.numpy as jnp
from jax import lax
from jax.experimental import pallas as pl
from jax.experimental.pallas import tpu as pltpu
from jax.experimental.pallas import tpu_sc as plsc

VOCAB = 100000
D_MODEL = 1024
MAX_LEN = 2048
BATCH = 4
SEQ = 2048

NC, NS = 2, 16
NW = NC * NS
TOTAL = BATCH * SEQ
PER_W = TOTAL // NW
CHUNK = 16
N_CHUNKS = PER_W // CHUNK
NR = 2
NO = 4
SUPER = 4
SCALE = float(D_MODEL) ** 0.5


def _make_pe(max_len, d_model):
    pe = np.zeros((max_len, d_model), dtype=np.float32)
    position = np.arange(0, max_len, dtype=np.float32)[:, None]
    div_term = np.exp(
        np.arange(0, d_model, 2, dtype=np.float32) * -(np.log(10000.0) / d_model))
    pe[:, 0::2] = np.sin(position * div_term)
    pe[:, 1::2] = np.cos(position * div_term)
    return pe


_PE = _make_pe(MAX_LEN, D_MODEL)


def _sc_embed(x_flat, table, pe):
    mesh = plsc.VectorSubcoreMesh(core_axis_name="c", subcore_axis_name="s")

    @functools.partial(
        pl.kernel,
        out_type=jax.ShapeDtypeStruct((TOTAL, D_MODEL), jnp.float32),
        mesh=mesh,
        scratch_types=[
            pltpu.VMEM((PER_W,), jnp.int32),
            [pltpu.VMEM((CHUNK, D_MODEL), jnp.float32) for _ in range(NR)],
            [pltpu.VMEM((CHUNK, D_MODEL), jnp.float32) for _ in range(NO)],
            [pltpu.SemaphoreType.DMA for _ in range(NR)],
            [pltpu.SemaphoreType.DMA for _ in range(NO)],
            [pltpu.SemaphoreType.DMA for _ in range(NO)],
        ],
    )
    def k(x_hbm, table_hbm, pe_hbm, out_hbm,
          idx_v, rows, obufs, gsems, psems, osems):
        wid = lax.axis_index("s") * NC + lax.axis_index("c")
        base = wid * PER_W
        s0 = base % SEQ

        pltpu.sync_copy(x_hbm.at[pl.ds(base, PER_W)], idx_v)

        def fire_gather(c, br):
            pltpu.async_copy(
                table_hbm.at[idx_v.at[pl.ds(c * CHUNK, CHUNK)]],
                rows[br], gsems[br])

        def wait_gather(c, br):
            pltpu.make_async_copy(
                table_hbm.at[idx_v.at[pl.ds(c * CHUNK, CHUNK)]],
                rows[br], gsems[br]).wait()

        def fire_pe(c, bo):
            pltpu.async_copy(
                pe_hbm.at[pl.ds(s0 + c * CHUNK, CHUNK)], obufs[bo], psems[bo])

        def wait_pe(c, bo):
            pltpu.make_async_copy(
                pe_hbm.at[pl.ds(s0 + c * CHUNK, CHUNK)], obufs[bo],
                psems[bo]).wait()

        def fire_out(c, bo):
            pltpu.async_copy(
                obufs[bo], out_hbm.at[pl.ds(base + c * CHUNK, CHUNK)],
                osems[bo])

        def wait_out(c, bo):
            pltpu.make_async_copy(
                obufs[bo], out_hbm.at[pl.ds(base + c * CHUNK, CHUNK)],
                osems[bo]).wait()

        fire_gather(0, 0)
        fire_gather(1, 1)
        fire_pe(0, 0)
        fire_pe(1, 1)

        def super_body(g, _):
            c0 = g * SUPER
            for j in range(SUPER):
                c = c0 + j
                br, bo = j % NR, j % NO

                @pl.when(c >= NO - 2)
                def _():
                    wait_out(c - (NO - 2), (j + 2) % NO)

                @pl.when(c + 2 < N_CHUNKS)
                def _():
                    fire_pe(c + 2, (j + 2) % NO)

                wait_gather(c, br)
                wait_pe(c, bo)

                @plsc.parallel_loop(0, CHUNK)
                def row_body(r):
                    for q in range(D_MODEL // 16):
                        sl = pl.ds(q * 16, 16)
                        plsc.addupdate(
                            obufs[bo].at[r, sl], rows[br][r, sl] * SCALE)

                fire_out(c, bo)

                @pl.when(c + 2 < N_CHUNKS)
                def _():
                    fire_gather(c + 2, br)
            return 0

        lax.fori_loop(0, N_CHUNKS // SUPER, super_body, 0)

        for c in range(N_CHUNKS - 2, N_CHUNKS):
            wait_out(c, c % NO)

    return k(x_flat, table, pe)


def kernel(x, table):
    x_flat = jnp.reshape(x, (TOTAL,)).astype(jnp.int32)
    out = _sc_embed(x_flat, table, _PE)
    return jnp.reshape(out, (BATCH, SEQ, D_MODEL))
